# baseline (speedup 1.0000x reference)
import functools

import jax
import jax.numpy as jnp
from jax import lax
from jax.experimental import pallas as pl
from jax.experimental.pallas import tpu as pltpu
from jax.experimental.pallas import tpu_sc as plsc

_R = 128
_NW = 32
_GCHUNK = 128


def _vq_tc_kernel(z_ref, zsq_ref, w_ref, idx_ref, loss_ref):
    i = pl.program_id(0)
    nsteps = pl.num_programs(0)
    z = z_ref[...]
    w = w_ref[...]
    kk = w.shape[0]
    half = kk // 2

    zb = z.astype(jnp.bfloat16).astype(jnp.float32)
    zsq = zsq_ref[...]
    wsq = jnp.sum(w * w, axis=1)
    mm = lax.dot_general(zb, w, (((1,), (1,)), ((), ())))
    d = (zsq + wsq) - 2.0 * mm

    d0 = d[:, :half]
    d1 = d[:, half:]
    m0 = jnp.min(d0, axis=1, keepdims=True)
    m1 = jnp.min(d1, axis=1, keepdims=True)
    io = lax.broadcasted_iota(jnp.int32, d0.shape, 1)
    i0 = jnp.min(jnp.where(d0 == m0, io, half), axis=1)
    i1 = jnp.min(jnp.where(d1 == m1, io, half), axis=1) + half
    mb = m0[:, 0].astype(jnp.bfloat16).astype(jnp.float32)
    take1 = m1[:, 0] < mb
    idx = jnp.where(take1, i1, i0)
    idx_ref[...] = idx.reshape(1, 1, _R)

    msel = jnp.where(take1, m1[:, 0], m0[:, 0])
    @pl.when(i == 0)
    def _init():
        loss_ref[...] = jnp.zeros((1, 1), jnp.float32)
    loss_ref[...] += jnp.sum(msel, keepdims=True).reshape(1, 1)
    @pl.when(i == nsteps - 1)
    def _fin():
        total = nsteps * _R * z.shape[1]
        loss_ref[...] = loss_ref[...] * (0.25 / total)


def _make_sc_gather(n, dim):
    bpw = n // _NW
    nch = bpw // _GCHUNK

    @functools.partial(
        pl.kernel,
        out_type=jax.ShapeDtypeStruct((n, dim), jnp.float32),
        mesh=plsc.VectorSubcoreMesh(core_axis_name="c", subcore_axis_name="s"),
        scratch_types=[
            pltpu.VMEM((bpw,), jnp.int32),
            pltpu.VMEM((bpw, dim), jnp.float32),
            pltpu.VMEM((bpw, dim), jnp.float32),
            pltpu.SemaphoreType.DMA,
        ],
        compiler_params=pltpu.CompilerParams(use_tc_tiling_on_sc=False),
    )
    def sc_gather_st(w_hbm, idx_hbm, z_hbm, out_hbm, idx_v, rows_v, z_v, sem):
        wid = lax.axis_index("s") * 2 + lax.axis_index("c")
        base = wid * bpw
        pltpu.sync_copy(idx_hbm.at[pl.ds(base, bpw)], idx_v)
        pltpu.sync_copy(z_hbm.at[pl.ds(base, bpw)], z_v)
        copies = []
        for j in range(nch):
            copies.append(pltpu.async_copy(
                w_hbm.at[idx_v.at[pl.ds(j * _GCHUNK, _GCHUNK)]],
                rows_v.at[pl.ds(j * _GCHUNK, _GCHUNK)],
                sem,
            ))
        for c in copies:
            c.wait()

        def row_body(r, carry):
            for h in range(dim // 16):
                zv = z_v[r, pl.ds(h * 16, 16)]
                rv = rows_v[r, pl.ds(h * 16, 16)]
                rows_v[r, pl.ds(h * 16, 16)] = zv + (rv - zv)
            return carry

        lax.fori_loop(0, bpw, row_body, 0)
        pltpu.sync_copy(rows_v, out_hbm.at[pl.ds(base, bpw)])

    return sc_gather_st


def kernel(z, W):
    Bz, Tz, Dz = z.shape
    Kz = W.shape[0]
    n = Bz * Tz
    nb = n // _R
    z_flat = z.reshape(n, Dz)
    zsq = jnp.sum(z_flat ** 2, axis=1, keepdims=True)

    idx3, loss = pl.pallas_call(
        _vq_tc_kernel,
        grid=(nb,),
        in_specs=[
            pl.BlockSpec((_R, Dz), lambda i: (i, 0)),
            pl.BlockSpec((_R, 1), lambda i: (i, 0)),
            pl.BlockSpec((Kz, Dz), lambda i: (0, 0)),
        ],
        out_specs=[
            pl.BlockSpec((1, 1, _R), lambda i: (i, 0, 0)),
            pl.BlockSpec((1, 1), lambda i: (0, 0)),
        ],
        out_shape=[
            jax.ShapeDtypeStruct((nb, 1, _R), jnp.int32),
            jax.ShapeDtypeStruct((1, 1), jnp.float32),
        ],
        compiler_params=pltpu.CompilerParams(
            dimension_semantics=("arbitrary",),
        ),
    )(z_flat, zsq, W)

    indices = idx3.reshape(n)
    zq_st = _make_sc_gather(n, Dz)(W, indices, z_flat)

    return (zq_st.reshape(Bz, Tz, Dz),
            indices.reshape(Bz, Tz),
            loss.reshape(()))


# speedup vs baseline: 1.3181x; 1.3181x over previous
import functools

import jax
import jax.numpy as jnp
from jax import lax
from jax.experimental import pallas as pl
from jax.experimental.pallas import tpu as pltpu
from jax.experimental.pallas import tpu_sc as plsc

_R = 128
_NW = 32
_GCHUNK = 128


def _vq_tc_kernel(z_ref, zsq_ref, w2_ref, wsq_ref, idx_ref, loss_ref):
    i = pl.program_id(0)
    nsteps = pl.num_programs(0)
    z = z_ref[...]
    w2 = w2_ref[...]
    kk = w2.shape[0]
    half = kk // 2

    zb = z.astype(jnp.bfloat16).astype(jnp.float32)
    zsq = zsq_ref[...]
    wsq = wsq_ref[...]
    mm2 = lax.dot_general(zb, w2, (((1,), (1,)), ((), ())))
    d = (zsq + wsq) - mm2

    d0 = d[:, :half]
    d1 = d[:, half:]
    m0 = jnp.min(d0, axis=1, keepdims=True)
    m1 = jnp.min(d1, axis=1, keepdims=True)
    io = lax.broadcasted_iota(jnp.int32, d0.shape, 1)
    i0 = jnp.min(jnp.where(d0 == m0, io, half), axis=1)
    i1 = jnp.min(jnp.where(d1 == m1, io, half), axis=1) + half
    mb = m0[:, 0].astype(jnp.bfloat16).astype(jnp.float32)
    take1 = m1[:, 0] < mb
    idx = jnp.where(take1, i1, i0)
    idx_ref[...] = idx.reshape(1, 1, _R)

    msel = jnp.where(take1, m1[:, 0], m0[:, 0])
    @pl.when(i == 0)
    def _init():
        loss_ref[...] = jnp.zeros((1, 1), jnp.float32)
    loss_ref[...] += jnp.sum(msel, keepdims=True).reshape(1, 1)
    @pl.when(i == nsteps - 1)
    def _fin():
        total = nsteps * _R * z.shape[1]
        loss_ref[...] = loss_ref[...] * (0.25 / total)


def _make_sc_gather(n, dim):
    bpw = n // _NW
    nch = bpw // _GCHUNK

    @functools.partial(
        pl.kernel,
        out_type=jax.ShapeDtypeStruct((n, dim), jnp.float32),
        mesh=plsc.VectorSubcoreMesh(core_axis_name="c", subcore_axis_name="s"),
        scratch_types=[
            pltpu.VMEM((bpw,), jnp.int32),
            pltpu.VMEM((bpw, dim), jnp.float32),
            pltpu.VMEM((bpw, dim), jnp.float32),
            pltpu.SemaphoreType.DMA,
        ],
        compiler_params=pltpu.CompilerParams(use_tc_tiling_on_sc=False),
    )
    def sc_gather_st(w_hbm, idx_hbm, z_hbm, out_hbm, idx_v, rows_v, z_v, sem):
        wid = lax.axis_index("s") * 2 + lax.axis_index("c")
        base = wid * bpw
        pltpu.sync_copy(idx_hbm.at[pl.ds(base, bpw)], idx_v)
        pltpu.sync_copy(z_hbm.at[pl.ds(base, bpw)], z_v)
        copies = []
        for j in range(nch):
            copies.append(pltpu.async_copy(
                w_hbm.at[idx_v.at[pl.ds(j * _GCHUNK, _GCHUNK)]],
                rows_v.at[pl.ds(j * _GCHUNK, _GCHUNK)],
                sem,
            ))
        for c in copies:
            c.wait()

        def row_body(r, carry):
            for h in range(dim // 16):
                zv = z_v[r, pl.ds(h * 16, 16)]
                rv = rows_v[r, pl.ds(h * 16, 16)]
                rows_v[r, pl.ds(h * 16, 16)] = zv + (rv - zv)
            return carry

        lax.fori_loop(0, bpw, row_body, 0)
        pltpu.sync_copy(rows_v, out_hbm.at[pl.ds(base, bpw)])

    return sc_gather_st


def kernel(z, W):
    Bz, Tz, Dz = z.shape
    Kz = W.shape[0]
    n = Bz * Tz
    nb = n // _R
    z_flat = z.reshape(n, Dz)
    zsq = jnp.sum(z_flat ** 2, axis=1, keepdims=True)
    wsq = jnp.sum(W ** 2, axis=1).reshape(1, Kz)
    w2 = W + W

    idx3, loss = pl.pallas_call(
        _vq_tc_kernel,
        grid=(nb,),
        in_specs=[
            pl.BlockSpec((_R, Dz), lambda i: (i, 0)),
            pl.BlockSpec((_R, 1), lambda i: (i, 0)),
            pl.BlockSpec((Kz, Dz), lambda i: (0, 0)),
            pl.BlockSpec((1, Kz), lambda i: (0, 0)),
        ],
        out_specs=[
            pl.BlockSpec((1, 1, _R), lambda i: (i, 0, 0)),
            pl.BlockSpec((1, 1), lambda i: (0, 0)),
        ],
        out_shape=[
            jax.ShapeDtypeStruct((nb, 1, _R), jnp.int32),
            jax.ShapeDtypeStruct((1, 1), jnp.float32),
        ],
        compiler_params=pltpu.CompilerParams(
            dimension_semantics=("arbitrary",),
        ),
    )(z_flat, zsq, w2, wsq)

    indices = idx3.reshape(n)
    zq_st = _make_sc_gather(n, Dz)(W, indices, z_flat)

    return (zq_st.reshape(Bz, Tz, Dz),
            indices.reshape(Bz, Tz),
            loss.reshape(()))


# speedup vs baseline: 1.5233x; 1.1557x over previous
import functools

import jax
import jax.numpy as jnp
from jax import lax
from jax.experimental import pallas as pl
from jax.experimental.pallas import tpu as pltpu
from jax.experimental.pallas import tpu_sc as plsc

_R = 256
_NW = 32
_GCHUNK = 128


def _vq_tc_kernel(z_ref, zsq_ref, w2_ref, wsq_ref, idx_ref, loss_ref):
    i = pl.program_id(0)
    nsteps = pl.num_programs(0)
    z = z_ref[...]
    w2 = w2_ref[...]
    kk = w2.shape[0]
    half = kk // 2

    zb = z.astype(jnp.bfloat16).astype(jnp.float32)
    zsq = zsq_ref[...]
    wsq = wsq_ref[...]
    mm2 = lax.dot_general(zb, w2, (((1,), (1,)), ((), ())))
    d = (zsq + wsq) - mm2

    d0 = d[:, :half]
    d1 = d[:, half:]
    m0 = jnp.min(d0, axis=1, keepdims=True)
    m1 = jnp.min(d1, axis=1, keepdims=True)
    io = lax.broadcasted_iota(jnp.int32, d0.shape, 1)
    i0 = jnp.min(jnp.where(d0 == m0, io, half), axis=1)
    i1 = jnp.min(jnp.where(d1 == m1, io, half), axis=1) + half
    mb = m0[:, 0].astype(jnp.bfloat16).astype(jnp.float32)
    take1 = m1[:, 0] < mb
    idx = jnp.where(take1, i1, i0)
    idx_ref[...] = idx.reshape(1, 1, _R)

    msel = jnp.where(take1, m1[:, 0], m0[:, 0])
    @pl.when(i == 0)
    def _init():
        loss_ref[...] = jnp.zeros((1, 1), jnp.float32)
    loss_ref[...] += jnp.sum(msel, keepdims=True).reshape(1, 1)
    @pl.when(i == nsteps - 1)
    def _fin():
        total = nsteps * _R * z.shape[1]
        loss_ref[...] = loss_ref[...] * (0.25 / total)


def _make_sc_gather(n, dim):
    bpw = n // _NW
    nch = bpw // _GCHUNK

    @functools.partial(
        pl.kernel,
        out_type=jax.ShapeDtypeStruct((n, dim), jnp.float32),
        mesh=plsc.VectorSubcoreMesh(core_axis_name="c", subcore_axis_name="s"),
        scratch_types=[
            pltpu.VMEM((bpw,), jnp.int32),
            pltpu.VMEM((bpw, dim), jnp.float32),
            pltpu.VMEM((bpw, dim), jnp.float32),
            pltpu.SemaphoreType.DMA,
        ],
        compiler_params=pltpu.CompilerParams(use_tc_tiling_on_sc=False),
    )
    def sc_gather_st(w_hbm, idx_hbm, z_hbm, out_hbm, idx_v, rows_v, z_v, sem):
        wid = lax.axis_index("s") * 2 + lax.axis_index("c")
        base = wid * bpw
        pltpu.sync_copy(idx_hbm.at[pl.ds(base, bpw)], idx_v)
        pltpu.sync_copy(z_hbm.at[pl.ds(base, bpw)], z_v)
        copies = []
        for j in range(nch):
            copies.append(pltpu.async_copy(
                w_hbm.at[idx_v.at[pl.ds(j * _GCHUNK, _GCHUNK)]],
                rows_v.at[pl.ds(j * _GCHUNK, _GCHUNK)],
                sem,
            ))
        for c in copies:
            c.wait()

        def row_body(r, carry):
            for h in range(dim // 16):
                zv = z_v[r, pl.ds(h * 16, 16)]
                rv = rows_v[r, pl.ds(h * 16, 16)]
                rows_v[r, pl.ds(h * 16, 16)] = zv + (rv - zv)
            return carry

        lax.fori_loop(0, bpw, row_body, 0)
        pltpu.sync_copy(rows_v, out_hbm.at[pl.ds(base, bpw)])

    return sc_gather_st


def kernel(z, W):
    Bz, Tz, Dz = z.shape
    Kz = W.shape[0]
    n = Bz * Tz
    nb = n // _R
    z_flat = z.reshape(n, Dz)
    zsq = jnp.sum(z_flat ** 2, axis=1, keepdims=True)
    wsq = jnp.sum(W ** 2, axis=1).reshape(1, Kz)
    w2 = W + W

    idx3, loss = pl.pallas_call(
        _vq_tc_kernel,
        grid=(nb,),
        in_specs=[
            pl.BlockSpec((_R, Dz), lambda i: (i, 0)),
            pl.BlockSpec((_R, 1), lambda i: (i, 0)),
            pl.BlockSpec((Kz, Dz), lambda i: (0, 0)),
            pl.BlockSpec((1, Kz), lambda i: (0, 0)),
        ],
        out_specs=[
            pl.BlockSpec((1, 1, _R), lambda i: (i, 0, 0)),
            pl.BlockSpec((1, 1), lambda i: (0, 0)),
        ],
        out_shape=[
            jax.ShapeDtypeStruct((nb, 1, _R), jnp.int32),
            jax.ShapeDtypeStruct((1, 1), jnp.float32),
        ],
        compiler_params=pltpu.CompilerParams(
            dimension_semantics=("arbitrary",),
        ),
    )(z_flat, zsq, w2, wsq)

    indices = idx3.reshape(n)
    zq_st = _make_sc_gather(n, Dz)(W, indices, z_flat)

    return (zq_st.reshape(Bz, Tz, Dz),
            indices.reshape(Bz, Tz),
            loss.reshape(()))


# speedup vs baseline: 1.5457x; 1.0147x over previous
import functools

import jax
import jax.numpy as jnp
from jax import lax
from jax.experimental import pallas as pl
from jax.experimental.pallas import tpu as pltpu
from jax.experimental.pallas import tpu_sc as plsc

_R = 512
_NW = 32
_GCHUNK = 128


def _vq_tc_kernel(z_ref, zsq_ref, w2_ref, wsq_ref, idx_ref, loss_ref):
    i = pl.program_id(0)
    nsteps = pl.num_programs(0)
    z = z_ref[...]
    w2 = w2_ref[...]
    kk = w2.shape[0]
    half = kk // 2

    zb = z.astype(jnp.bfloat16).astype(jnp.float32)
    zsq = zsq_ref[...]
    wsq = wsq_ref[...]
    mm2 = lax.dot_general(zb, w2, (((1,), (1,)), ((), ())))
    d = (zsq + wsq) - mm2

    d0 = d[:, :half]
    d1 = d[:, half:]
    m0 = jnp.min(d0, axis=1, keepdims=True)
    m1 = jnp.min(d1, axis=1, keepdims=True)
    io = lax.broadcasted_iota(jnp.int32, d0.shape, 1)
    i0 = jnp.min(jnp.where(d0 == m0, io, half), axis=1)
    i1 = jnp.min(jnp.where(d1 == m1, io, half), axis=1) + half
    mb = m0[:, 0].astype(jnp.bfloat16).astype(jnp.float32)
    take1 = m1[:, 0] < mb
    idx = jnp.where(take1, i1, i0)
    idx_ref[...] = idx.reshape(1, 1, _R)

    msel = jnp.where(take1, m1[:, 0], m0[:, 0])
    @pl.when(i == 0)
    def _init():
        loss_ref[...] = jnp.zeros((1, 1), jnp.float32)
    loss_ref[...] += jnp.sum(msel, keepdims=True).reshape(1, 1)
    @pl.when(i == nsteps - 1)
    def _fin():
        total = nsteps * _R * z.shape[1]
        loss_ref[...] = loss_ref[...] * (0.25 / total)


def _make_sc_gather(n, dim):
    bpw = n // _NW
    nch = bpw // _GCHUNK

    @functools.partial(
        pl.kernel,
        out_type=jax.ShapeDtypeStruct((n, dim), jnp.float32),
        mesh=plsc.VectorSubcoreMesh(core_axis_name="c", subcore_axis_name="s"),
        scratch_types=[
            pltpu.VMEM((bpw,), jnp.int32),
            pltpu.VMEM((bpw, dim), jnp.float32),
            pltpu.VMEM((bpw, dim), jnp.float32),
            pltpu.SemaphoreType.DMA,
        ],
        compiler_params=pltpu.CompilerParams(use_tc_tiling_on_sc=False),
    )
    def sc_gather_st(w_hbm, idx_hbm, z_hbm, out_hbm, idx_v, rows_v, z_v, sem):
        wid = lax.axis_index("s") * 2 + lax.axis_index("c")
        base = wid * bpw
        pltpu.sync_copy(idx_hbm.at[pl.ds(base, bpw)], idx_v)
        pltpu.sync_copy(z_hbm.at[pl.ds(base, bpw)], z_v)
        copies = []
        for j in range(nch):
            copies.append(pltpu.async_copy(
                w_hbm.at[idx_v.at[pl.ds(j * _GCHUNK, _GCHUNK)]],
                rows_v.at[pl.ds(j * _GCHUNK, _GCHUNK)],
                sem,
            ))
        for c in copies:
            c.wait()

        def row_body(r, carry):
            for h in range(dim // 16):
                zv = z_v[r, pl.ds(h * 16, 16)]
                rv = rows_v[r, pl.ds(h * 16, 16)]
                rows_v[r, pl.ds(h * 16, 16)] = zv + (rv - zv)
            return carry

        lax.fori_loop(0, bpw, row_body, 0)
        pltpu.sync_copy(rows_v, out_hbm.at[pl.ds(base, bpw)])

    return sc_gather_st


def kernel(z, W):
    Bz, Tz, Dz = z.shape
    Kz = W.shape[0]
    n = Bz * Tz
    nb = n // _R
    z_flat = z.reshape(n, Dz)
    zsq = jnp.sum(z_flat ** 2, axis=1, keepdims=True)
    wsq = jnp.sum(W ** 2, axis=1).reshape(1, Kz)
    w2 = W + W

    idx3, loss = pl.pallas_call(
        _vq_tc_kernel,
        grid=(nb,),
        in_specs=[
            pl.BlockSpec((_R, Dz), lambda i: (i, 0)),
            pl.BlockSpec((_R, 1), lambda i: (i, 0)),
            pl.BlockSpec((Kz, Dz), lambda i: (0, 0)),
            pl.BlockSpec((1, Kz), lambda i: (0, 0)),
        ],
        out_specs=[
            pl.BlockSpec((1, 1, _R), lambda i: (i, 0, 0)),
            pl.BlockSpec((1, 1), lambda i: (0, 0)),
        ],
        out_shape=[
            jax.ShapeDtypeStruct((nb, 1, _R), jnp.int32),
            jax.ShapeDtypeStruct((1, 1), jnp.float32),
        ],
        compiler_params=pltpu.CompilerParams(
            dimension_semantics=("arbitrary",),
        ),
    )(z_flat, zsq, w2, wsq)

    indices = idx3.reshape(n)
    zq_st = _make_sc_gather(n, Dz)(W, indices, z_flat)

    return (zq_st.reshape(Bz, Tz, Dz),
            indices.reshape(Bz, Tz),
            loss.reshape(()))
